# baseline (speedup 1.0000x reference)
import functools

import jax
import jax.numpy as jnp
import numpy as np
from jax import lax
from jax.experimental import pallas as pl
from jax.experimental.pallas import tpu as pltpu
from jax.experimental.pallas import tpu_sc as plsc

N = 10000
E = 160000
B = 16
MAXLEN = 625
D = 256
DE = 16
MR = 4
H = 4
NUM_LAYERS = 4
DELTA = 0.1

_NC = 2
_NS = 16
_NW = _NC * _NS


def _bdot(a, b, out=jnp.float32):
    return jnp.dot(a.astype(jnp.bfloat16), b.astype(jnp.bfloat16),
                   preferred_element_type=out)


def _pack_bf16_pairs(x):
    bits = lax.bitcast_convert_type(x, jnp.int32)
    r = (bits + 0x7FFF + ((bits >> 16) & 1)) >> 16
    return (r[:, :128] & 0xFFFF) | (r[:, 128:] << 16)


def _unpack_bf16_pairs(p):
    lo = lax.bitcast_convert_type(p << 16, jnp.float32)
    hi = lax.bitcast_convert_type(p & jnp.int32(-65536), jnp.float32)
    return lo, hi


def _ln(x, g=None, b=None):
    m = jnp.mean(x, axis=-1, keepdims=True)
    v = jnp.mean((x - m) ** 2, axis=-1, keepdims=True)
    y = (x - m) * lax.rsqrt(v + 1e-5)
    if g is not None:
        y = y * g + b
    return y


def _gather_body(table_hbm, idx_hbm, out_hbm, idx_v, rows_v, sem, *,
                 per_w, chunk, n_ch):
    wid = lax.axis_index("s") * _NC + lax.axis_index("c")
    base = wid * per_w

    def body(j, carry):
        off = base + j * chunk
        pltpu.sync_copy(idx_hbm.at[pl.ds(off, chunk)], idx_v)
        pltpu.async_copy(table_hbm.at[idx_v], rows_v, sem).wait()
        pltpu.sync_copy(rows_v, out_hbm.at[pl.ds(off, chunk)])
        return carry

    lax.fori_loop(0, n_ch, body, 0)


def _sc_gather(table, idx, chunk):
    rows, dt = table.shape
    e = idx.shape[0]
    per_w = e // _NW
    assert per_w * _NW == e and per_w % chunk == 0 and chunk % 8 == 0
    n_ch = per_w // chunk
    mesh = plsc.VectorSubcoreMesh(core_axis_name="c", subcore_axis_name="s")
    f = pl.kernel(
        functools.partial(_gather_body, per_w=per_w, chunk=chunk, n_ch=n_ch),
        mesh=mesh,
        out_type=jax.ShapeDtypeStruct((e, dt), table.dtype),
        scratch_types=[
            pltpu.VMEM((chunk,), jnp.int32),
            pltpu.VMEM((chunk, dt), table.dtype),
            pltpu.SemaphoreType.DMA,
        ],
    )
    return f(table, idx)


def _scatter_body(msg_hbm, dst_hbm, init_hbm, out_hbm, idx_v, buf_v, acc,
                  *, ec, n_ch, per_tile, rows_per_tile):
    cid = lax.axis_index("c")
    sid = lax.axis_index("s")
    nbase = pl.multiple_of(sid * 624, 8)
    pltpu.sync_copy(init_hbm.at[cid, pl.ds(nbase, 640)], acc.at[pl.ds(nbase, 640)])
    plsc.subcore_barrier()
    ebase = sid * per_tile

    def body(j, carry):
        off = pl.multiple_of(ebase + j * ec, 8)
        pltpu.sync_copy(dst_hbm.at[pl.ds(off, ec)], idx_v)
        pltpu.sync_copy(msg_hbm.at[cid, pl.ds(off, ec)], buf_v)
        pltpu.sync_copy(buf_v, acc.at[idx_v], add=True)
        return carry

    lax.fori_loop(0, n_ch, body, 0)
    plsc.subcore_barrier()
    pltpu.sync_copy(acc.at[pl.ds(nbase, 640)],
                    out_hbm.at[cid, pl.ds(nbase, 640)])


def _sc_scatter_add(msg3, dst, init, ec=200):
    e = msg3.shape[1]
    per_tile = e // _NS
    rows_per_tile = N // _NS
    assert per_tile % ec == 0 and ec % 8 == 0
    assert 624 * (_NS - 1) + 640 == N
    n_ch = per_tile // ec
    mesh = plsc.VectorSubcoreMesh(core_axis_name="c", subcore_axis_name="s")
    f = pl.kernel(
        functools.partial(_scatter_body, ec=ec, n_ch=n_ch, per_tile=per_tile,
                          rows_per_tile=rows_per_tile),
        mesh=mesh,
        out_type=jax.ShapeDtypeStruct((2, N, 128), jnp.float32),
        scratch_types=[
            pltpu.VMEM((ec,), jnp.int32),
            pltpu.VMEM((ec, 128), jnp.float32),
            pltpu.VMEM_SHARED((N, 128), jnp.float32),
        ],
    )
    return f(msg3, dst, init)


_EB = 1600


def _edge_msg_body(hs_ref, bd_ref, he_ref, wa, wc, we2, wm1h, wm1e, wm2,
                   he_out, msg_out):
    hs_lo, hs_hi = _unpack_bf16_pairs(hs_ref[...])
    he = he_ref[...]
    f32 = jnp.float32
    z = (jnp.dot(hs_lo, wa[...][:128], preferred_element_type=f32)
         + jnp.dot(hs_hi, wa[...][128:], preferred_element_type=f32)
         + bd_ref[...][:, :MR * DE]
         + jnp.dot(he, wc[...], preferred_element_type=f32))
    m = jnp.dot(jax.nn.relu(z), we2[...], preferred_element_type=f32)
    he2 = _ln(he + m)
    he_out[...] = he2
    p = jax.nn.relu(
        jnp.dot(hs_lo, wm1h[...][:128], preferred_element_type=f32)
        + jnp.dot(hs_hi, wm1h[...][128:], preferred_element_type=f32)
        + jnp.dot(he2, wm1e[...], preferred_element_type=f32))
    msgv = jnp.dot(p, wm2[...], preferred_element_type=f32)
    msg_out[0] = msgv[:, :128]
    msg_out[1] = msgv[:, 128:]


def _tc_edge_msg(hs, bd, he, wa, wc, we2, wm1h, wm1e, wm2):
    e = hs.shape[0]
    grid = (e // _EB,)
    full = lambda *s: pl.BlockSpec(s, lambda i: (0,) * len(s))
    row = lambda *s: pl.BlockSpec(s, lambda i: (i,) + (0,) * (len(s) - 1))
    return pl.pallas_call(
        _edge_msg_body,
        grid=grid,
        in_specs=[
            row(_EB, 128), row(_EB, 128), row(_EB, DE),
            full(D, MR * DE), full(DE, MR * DE), full(MR * DE, DE),
            full(D, MR * D), full(DE, MR * D), full(MR * D, D),
        ],
        out_specs=[row(_EB, DE),
                   pl.BlockSpec((2, _EB, 128), lambda i: (0, i, 0))],
        out_shape=[
            jax.ShapeDtypeStruct((e, DE), jnp.float32),
            jax.ShapeDtypeStruct((2, e, 128), jnp.float32),
        ],
    )(hs, bd, he, wa, wc, we2, wm1h, wm1e, wm2)


_NB = 2000


def _node_body(h_ref, agg_ref, wu1h, wu1a, wu2, wbn, h_out, hb_out, btab_out):
    h = h_ref[...]
    f32 = jnp.float32
    u0 = jax.nn.relu(jnp.dot(h, wu1h[...], preferred_element_type=f32)
                     + jnp.dot(agg_ref[0], wu1a[...][:128],
                               preferred_element_type=f32)
                     + jnp.dot(agg_ref[1], wu1a[...][128:],
                               preferred_element_type=f32))
    u = jnp.dot(u0, wu2[...], preferred_element_type=f32)
    h2 = _ln(h + DELTA * u)
    h_out[...] = h2
    hb_out[...] = _pack_bf16_pairs(h2)
    btab_out[...] = jnp.dot(h2, wbn[...], preferred_element_type=jnp.float32)


def _tc_node_update(h, agg, wu1h, wu1a, wu2, wbn):
    grid = (N // _NB,)
    full = lambda *s: pl.BlockSpec(s, lambda i: (0,) * len(s))
    row = lambda *s: pl.BlockSpec(s, lambda i: (i,) + (0,) * (len(s) - 1))
    return pl.pallas_call(
        _node_body,
        grid=grid,
        in_specs=[
            row(_NB, D),
            pl.BlockSpec((2, _NB, 128), lambda i: (0, i, 0)),
            full(D, MR * D), full(D, MR * D), full(MR * D, D),
            full(D, 128),
        ],
        out_specs=[row(_NB, D), row(_NB, 128), row(_NB, 128)],
        out_shape=[
            jax.ShapeDtypeStruct((N, D), jnp.float32),
            jax.ShapeDtypeStruct((N, 128), jnp.int32),
            jax.ShapeDtypeStruct((N, 128), jnp.float32),
        ],
    )(h, agg, wu1h, wu1a, wu2, wbn)


def _btab_body(h_ref, wb, out_ref, hb_ref):
    out_ref[...] = jnp.dot(h_ref[...], wb[...],
                           preferred_element_type=jnp.float32)
    hb_ref[...] = _pack_bf16_pairs(h_ref[...])


def _tc_btab(h, wb):
    return pl.pallas_call(
        _btab_body,
        grid=(N // _NB,),
        in_specs=[pl.BlockSpec((_NB, D), lambda i: (i, 0)),
                  pl.BlockSpec((D, 128), lambda i: (0, 0))],
        out_specs=[pl.BlockSpec((_NB, 128), lambda i: (i, 0)),
                   pl.BlockSpec((_NB, 128), lambda i: (i, 0))],
        out_shape=[jax.ShapeDtypeStruct((N, 128), jnp.float32),
                   jax.ShapeDtypeStruct((N, 128), jnp.int32)],
    )(h, wb)


def _tf_body(bnn_ref, x_ref, pn_g, pn_b, wq, bq, wk, bk, wv, bv, wo, bo,
             wf1, bf1, wf2, bf2, l1g, l1b, l2g, l2b, wl, bl, lng, lnb,
             lf_ref, g_ref):
    bidx = pl.program_id(0)
    nb = bnn_ref[bidx]
    x0 = _ln(x_ref[0], pn_g[...], pn_b[...])
    q = _bdot(x0, wq[...]) + bq[...]
    k = _bdot(x0, wk[...]) + bk[...]
    v = _bdot(x0, wv[...]) + bv[...]
    colmask = lax.broadcasted_iota(jnp.int32, (MAXLEN, MAXLEN), 1) >= nb
    dh = D // H
    outs = []
    for hh in range(H):
        sl = slice(hh * dh, (hh + 1) * dh)
        s = lax.dot_general(q[:, sl].astype(jnp.bfloat16),
                            k[:, sl].astype(jnp.bfloat16),
                            (((1,), (1,)), ((), ())),
                            preferred_element_type=jnp.float32)
        s = s * np.float32(1.0 / np.sqrt(dh))
        s = jnp.where(colmask, np.float32(-1e9), s)
        s = s - jnp.max(s, axis=-1, keepdims=True)
        es = jnp.exp(s)
        a = es / jnp.sum(es, axis=-1, keepdims=True)
        outs.append(_bdot(a, v[:, sl]))
    o = jnp.concatenate(outs, axis=1)
    o = _bdot(o, wo[...]) + bo[...]
    x1 = _ln(x0 + o, l1g[...], l1b[...])
    f0 = jax.nn.relu(_bdot(x1, wf1[...]) + bf1[...])
    f = _bdot(f0, wf2[...]) + bf2[...]
    lf = _ln(x1 + f, l2g[...], l2b[...])
    lf_ref[0] = lf
    rowmask = lax.broadcasted_iota(jnp.int32, (MAXLEN, 1), 0) < nb
    pooled = (jnp.sum(jnp.where(rowmask, lf, 0.0), axis=0, keepdims=True)
              / nb.astype(jnp.float32))
    g = _ln(_bdot(pooled, wl[...]) + bl[...], lng[...], lnb[...])
    g_ref[0] = g


def _tc_transformer(h, bnn, tf, wl, bl, lng, lnb, pn_g, pn_b):
    xb = h.reshape(B, MAXLEN, D)
    r2 = lambda a: a.reshape(1, -1)
    full = lambda *s: pl.BlockSpec(s, lambda i: (0,) * len(s))
    args = [
        xb, r2(pn_g), r2(pn_b),
        tf["Wq"], r2(tf["bq"]), tf["Wk"], r2(tf["bk"]),
        tf["Wv"], r2(tf["bv"]), tf["Wo"], r2(tf["bo"]),
        tf["Wf1"], r2(tf["bf1"]), tf["Wf2"], r2(tf["bf2"]),
        r2(tf["ln1_g"]), r2(tf["ln1_b"]), r2(tf["ln2_g"]), r2(tf["ln2_b"]),
        wl, r2(bl), r2(lng), r2(lnb),
    ]
    in_specs = [pl.BlockSpec(memory_space=pltpu.SMEM),
                pl.BlockSpec((1, MAXLEN, D), lambda i: (i, 0, 0))]
    in_specs += [full(*a.shape) for a in args[1:]]
    lf, g = pl.pallas_call(
        _tf_body,
        grid=(B,),
        in_specs=in_specs,
        out_specs=[pl.BlockSpec((1, MAXLEN, D), lambda i: (i, 0, 0)),
                   pl.BlockSpec((1, 1, D), lambda i: (i, 0, 0))],
        out_shape=[
            jax.ShapeDtypeStruct((B, MAXLEN, D), jnp.float32),
            jax.ShapeDtypeStruct((B, 1, D), jnp.float32),
        ],
    )(bnn, *args)
    return lf.reshape(N, D), g.reshape(B, D)


_EA = 83200


def kernel(h, he, edge_index, batch_num_nodes, params):
    src = edge_index[0]
    dst = edge_index[1]
    halves = [
        (src[:_EA], dst[:_EA], he[:_EA]),
        (src[_EA:], dst[_EA:], he[_EA:]),
    ]
    zero_agg = jnp.zeros((2, N, 128), jnp.float32)

    pad_b = lambda w: jnp.pad(w, ((0, 0), (0, 128 - MR * DE)))
    wb_next = pad_b(params["edge1"]["We1"][D:2 * D])
    btab, h_bf = _tc_btab(h, pad_b(params["edge0"]["We1"][D:2 * D]))

    for l in range(NUM_LAYERS):
        pe = params["edge0" if l == 0 else "edge1"]
        pn = params["node0" if l == 0 else "node1"]
        wa = pe["We1"][:D]
        wc = pe["We1"][2 * D:]
        we2 = pe["We2"]
        wm1h = pn["Wm1"][:D]
        wm1e = pn["Wm1"][D:]
        wu1h = pn["Wu1"][:D]
        wu1a = pn["Wu1"][D:]

        msgs = []
        new_halves = []
        for (src_h, dst_h, he_h) in halves:
            hs = _sc_gather(h_bf, src_h, chunk=200)
            bd = _sc_gather(btab, dst_h, chunk=200)
            he2, msg = _tc_edge_msg(hs, bd, he_h, wa, wc, we2=we2,
                                    wm1h=wm1h, wm1e=wm1e, wm2=pn["Wm2"])
            new_halves.append((src_h, dst_h, he2))
            msgs.append(msg)
        halves = new_halves
        agg = _sc_scatter_add(msgs[0], halves[0][1], zero_agg)
        agg = _sc_scatter_add(msgs[1], halves[1][1], agg)
        h, h_bf, btab = _tc_node_update(h, agg, wu1h, wu1a, pn["Wu2"],
                                        wb_next)

    local_feat, global_feat = _tc_transformer(
        h, batch_num_nodes, params["tf"], params["Wl"], params["bl"],
        params["ln_g"], params["ln_b"], params["pn_g"], params["pn_b"])
    return local_feat, global_feat


# speedup vs baseline: 1.3456x; 1.3456x over previous
import functools

import jax
import jax.numpy as jnp
import numpy as np
from jax import lax
from jax.experimental import pallas as pl
from jax.experimental.pallas import tpu as pltpu
from jax.experimental.pallas import tpu_sc as plsc

N = 10000
E = 160000
B = 16
MAXLEN = 625
D = 256
DE = 16
MR = 4
H = 4
NUM_LAYERS = 4
DELTA = 0.1

_NC = 2
_NS = 16
_NW = _NC * _NS


def _bdot(a, b, out=jnp.float32):
    return jnp.dot(a.astype(jnp.bfloat16), b.astype(jnp.bfloat16),
                   preferred_element_type=out)


def _pack_bf16_pairs(x):
    bits = lax.bitcast_convert_type(x, jnp.int32)
    r = (bits + 0x7FFF + ((bits >> 16) & 1)) >> 16
    return (r[:, :128] & 0xFFFF) | (r[:, 128:] << 16)


def _unpack_bf16_pairs(p):
    lo = lax.bitcast_convert_type(p << 16, jnp.float32)
    hi = lax.bitcast_convert_type(p & jnp.int32(-65536), jnp.float32)
    return lo, hi


def _ln(x, g=None, b=None):
    m = jnp.mean(x, axis=-1, keepdims=True)
    v = jnp.mean((x - m) ** 2, axis=-1, keepdims=True)
    y = (x - m) * lax.rsqrt(v + 1e-5)
    if g is not None:
        y = y * g + b
    return y


def _gather_body(table_hbm, idx_hbm, out_hbm, idx_v, rows0, rows1,
                 gs0, gs1, ws0, ws1, *, per_w, chunk, n_ch):
    wid = lax.axis_index("s") * _NC + lax.axis_index("c")
    base = wid * per_w
    pltpu.sync_copy(idx_hbm.at[pl.ds(base, per_w)], idx_v)
    rows = (rows0, rows1)
    gsem = (gs0, gs1)
    wsem = (ws0, ws1)

    def start_gather(j):
        return pltpu.async_copy(
            table_hbm.at[idx_v.at[pl.ds(j * chunk, chunk)]],
            rows[j % 2], gsem[j % 2])

    gh = {0: start_gather(0)}
    wh = {}
    for j in range(n_ch):
        if j >= 1:
            wh[j - 1].wait()
        if j + 1 < n_ch:
            gh[j + 1] = start_gather(j + 1)
        gh[j].wait()
        off = pl.multiple_of(base + j * chunk, 8)
        wh[j] = pltpu.async_copy(rows[j % 2], out_hbm.at[pl.ds(off, chunk)],
                                 wsem[j % 2])
    wh[n_ch - 1].wait()


def _sc_gather(table, idx, chunk):
    rows, dt = table.shape
    e = idx.shape[0]
    per_w = e // _NW
    assert per_w * _NW == e and per_w % chunk == 0 and chunk % 8 == 0
    n_ch = per_w // chunk
    mesh = plsc.VectorSubcoreMesh(core_axis_name="c", subcore_axis_name="s")
    f = pl.kernel(
        functools.partial(_gather_body, per_w=per_w, chunk=chunk, n_ch=n_ch),
        mesh=mesh,
        out_type=jax.ShapeDtypeStruct((e, dt), table.dtype),
        scratch_types=[
            pltpu.VMEM((per_w,), jnp.int32),
            pltpu.VMEM((chunk, dt), table.dtype),
            pltpu.VMEM((chunk, dt), table.dtype),
            pltpu.SemaphoreType.DMA,
            pltpu.SemaphoreType.DMA,
            pltpu.SemaphoreType.DMA,
            pltpu.SemaphoreType.DMA,
        ],
    )
    return f(table, idx)


def _scatter_body(msg_hbm, dst_hbm, init_hbm, out_hbm, idx0, idx1, buf0,
                  buf1, acc, is0, is1, ms0, ms1,
                  *, ec, n_ch, per_tile, rows_per_tile):
    cid = lax.axis_index("c")
    sid = lax.axis_index("s")
    nbase = pl.multiple_of(sid * 624, 8)
    pltpu.sync_copy(init_hbm.at[cid, pl.ds(nbase, 640)],
                    acc.at[pl.ds(nbase, 640)])
    plsc.subcore_barrier()
    ebase = sid * per_tile
    idx = (idx0, idx1)
    buf = (buf0, buf1)
    isem = (is0, is1)
    msem = (ms0, ms1)

    def start_loads(j):
        off = pl.multiple_of(ebase + j * ec, 8)
        return (pltpu.async_copy(dst_hbm.at[pl.ds(off, ec)], idx[j % 2],
                                 isem[j % 2]),
                pltpu.async_copy(msg_hbm.at[cid, pl.ds(off, ec)], buf[j % 2],
                                 msem[j % 2]))

    hs = {0: start_loads(0)}
    for j in range(n_ch):
        if j + 1 < n_ch:
            hs[j + 1] = start_loads(j + 1)
        hs[j][0].wait()
        hs[j][1].wait()
        pltpu.sync_copy(buf[j % 2], acc.at[idx[j % 2]], add=True)
    plsc.subcore_barrier()
    pltpu.sync_copy(acc.at[pl.ds(nbase, 640)],
                    out_hbm.at[cid, pl.ds(nbase, 640)])


def _sc_scatter_add(msg3, dst, init, ec=80):
    e = msg3.shape[1]
    per_tile = e // _NS
    rows_per_tile = N // _NS
    assert per_tile % ec == 0 and ec % 8 == 0
    assert 624 * (_NS - 1) + 640 == N
    n_ch = per_tile // ec
    mesh = plsc.VectorSubcoreMesh(core_axis_name="c", subcore_axis_name="s")
    f = pl.kernel(
        functools.partial(_scatter_body, ec=ec, n_ch=n_ch, per_tile=per_tile,
                          rows_per_tile=rows_per_tile),
        mesh=mesh,
        out_type=jax.ShapeDtypeStruct((2, N, 128), jnp.float32),
        scratch_types=[
            pltpu.VMEM((ec,), jnp.int32),
            pltpu.VMEM((ec,), jnp.int32),
            pltpu.VMEM((ec, 128), jnp.float32),
            pltpu.VMEM((ec, 128), jnp.float32),
            pltpu.VMEM_SHARED((N, 128), jnp.float32),
            pltpu.SemaphoreType.DMA,
            pltpu.SemaphoreType.DMA,
            pltpu.SemaphoreType.DMA,
            pltpu.SemaphoreType.DMA,
        ],
    )
    return f(msg3, dst, init)


_EB = 1600


def _edge_msg_body(hs_ref, bd_ref, he_ref, wa, wc, we2, wm1h, wm1e, wm2,
                   he_out, msg_out):
    hs_lo, hs_hi = _unpack_bf16_pairs(hs_ref[...])
    hs = jnp.concatenate([hs_lo, hs_hi], axis=-1)
    he = he_ref[...]
    f32 = jnp.float32
    z = (jnp.dot(hs, wa[...], preferred_element_type=f32)
         + bd_ref[...][:, :MR * DE]
         + jnp.dot(he, wc[...], preferred_element_type=f32))
    m = jnp.dot(jax.nn.relu(z), we2[...], preferred_element_type=f32)
    he2 = _ln(he + m)
    he_out[...] = he2
    p = jax.nn.relu(
        jnp.dot(hs, wm1h[...], preferred_element_type=f32)
        + jnp.dot(he2, wm1e[...], preferred_element_type=f32))
    msgv = jnp.dot(p, wm2[...], preferred_element_type=f32)
    msg_out[0] = msgv[:, :128]
    msg_out[1] = msgv[:, 128:]


def _tc_edge_msg(hs, bd, he, wa, wc, we2, wm1h, wm1e, wm2):
    e = hs.shape[0]
    grid = (e // _EB,)
    full = lambda *s: pl.BlockSpec(s, lambda i: (0,) * len(s))
    row = lambda *s: pl.BlockSpec(s, lambda i: (i,) + (0,) * (len(s) - 1))
    return pl.pallas_call(
        _edge_msg_body,
        grid=grid,
        in_specs=[
            row(_EB, 128), row(_EB, 128), row(_EB, DE),
            full(D, MR * DE), full(DE, MR * DE), full(MR * DE, DE),
            full(D, MR * D), full(DE, MR * D), full(MR * D, D),
        ],
        out_specs=[row(_EB, DE),
                   pl.BlockSpec((2, _EB, 128), lambda i: (0, i, 0))],
        out_shape=[
            jax.ShapeDtypeStruct((e, DE), jnp.float32),
            jax.ShapeDtypeStruct((2, e, 128), jnp.float32),
        ],
    )(hs, bd, he, wa, wc, we2, wm1h, wm1e, wm2)


_NB = 2000


def _node_body(h_ref, agg_ref, wu1h, wu1a, wu2, wbn, h_out, hb_out, btab_out):
    h = h_ref[...]
    f32 = jnp.float32
    u0 = jax.nn.relu(jnp.dot(h, wu1h[...], preferred_element_type=f32)
                     + jnp.dot(agg_ref[0], wu1a[...][:128],
                               preferred_element_type=f32)
                     + jnp.dot(agg_ref[1], wu1a[...][128:],
                               preferred_element_type=f32))
    u = jnp.dot(u0, wu2[...], preferred_element_type=f32)
    h2 = _ln(h + DELTA * u)
    h_out[...] = h2
    hb_out[...] = _pack_bf16_pairs(h2)
    btab_out[...] = jnp.dot(h2, wbn[...], preferred_element_type=jnp.float32)


def _tc_node_update(h, agg, wu1h, wu1a, wu2, wbn):
    grid = (N // _NB,)
    full = lambda *s: pl.BlockSpec(s, lambda i: (0,) * len(s))
    row = lambda *s: pl.BlockSpec(s, lambda i: (i,) + (0,) * (len(s) - 1))
    return pl.pallas_call(
        _node_body,
        grid=grid,
        in_specs=[
            row(_NB, D),
            pl.BlockSpec((2, _NB, 128), lambda i: (0, i, 0)),
            full(D, MR * D), full(D, MR * D), full(MR * D, D),
            full(D, 128),
        ],
        out_specs=[row(_NB, D), row(_NB, 128), row(_NB, 128)],
        out_shape=[
            jax.ShapeDtypeStruct((N, D), jnp.float32),
            jax.ShapeDtypeStruct((N, 128), jnp.int32),
            jax.ShapeDtypeStruct((N, 128), jnp.float32),
        ],
    )(h, agg, wu1h, wu1a, wu2, wbn)


def _btab_body(h_ref, wb, out_ref, hb_ref):
    out_ref[...] = jnp.dot(h_ref[...], wb[...],
                           preferred_element_type=jnp.float32)
    hb_ref[...] = _pack_bf16_pairs(h_ref[...])


def _tc_btab(h, wb):
    return pl.pallas_call(
        _btab_body,
        grid=(N // _NB,),
        in_specs=[pl.BlockSpec((_NB, D), lambda i: (i, 0)),
                  pl.BlockSpec((D, 128), lambda i: (0, 0))],
        out_specs=[pl.BlockSpec((_NB, 128), lambda i: (i, 0)),
                   pl.BlockSpec((_NB, 128), lambda i: (i, 0))],
        out_shape=[jax.ShapeDtypeStruct((N, 128), jnp.float32),
                   jax.ShapeDtypeStruct((N, 128), jnp.int32)],
    )(h, wb)


def _tf_body(bnn_ref, x_ref, pn_g, pn_b, wq, bq, wk, bk, wv, bv, wo, bo,
             wf1, bf1, wf2, bf2, l1g, l1b, l2g, l2b, wl, bl, lng, lnb,
             lf_ref, g_ref):
    bidx = pl.program_id(0)
    nb = bnn_ref[bidx]
    x0 = _ln(x_ref[0], pn_g[...], pn_b[...])
    q = _bdot(x0, wq[...]) + bq[...]
    k = _bdot(x0, wk[...]) + bk[...]
    v = _bdot(x0, wv[...]) + bv[...]
    colmask = lax.broadcasted_iota(jnp.int32, (MAXLEN, MAXLEN), 1) >= nb
    dh = D // H
    outs = []
    for hh in range(H):
        sl = slice(hh * dh, (hh + 1) * dh)
        s = lax.dot_general(q[:, sl].astype(jnp.bfloat16),
                            k[:, sl].astype(jnp.bfloat16),
                            (((1,), (1,)), ((), ())),
                            preferred_element_type=jnp.float32)
        s = s * np.float32(1.0 / np.sqrt(dh))
        s = jnp.where(colmask, np.float32(-1e9), s)
        s = s - jnp.max(s, axis=-1, keepdims=True)
        es = jnp.exp(s)
        a = es / jnp.sum(es, axis=-1, keepdims=True)
        outs.append(_bdot(a, v[:, sl]))
    o = jnp.concatenate(outs, axis=1)
    o = _bdot(o, wo[...]) + bo[...]
    x1 = _ln(x0 + o, l1g[...], l1b[...])
    f0 = jax.nn.relu(_bdot(x1, wf1[...]) + bf1[...])
    f = _bdot(f0, wf2[...]) + bf2[...]
    lf = _ln(x1 + f, l2g[...], l2b[...])
    lf_ref[0] = lf
    rowmask = lax.broadcasted_iota(jnp.int32, (MAXLEN, 1), 0) < nb
    pooled = (jnp.sum(jnp.where(rowmask, lf, 0.0), axis=0, keepdims=True)
              / nb.astype(jnp.float32))
    g = _ln(_bdot(pooled, wl[...]) + bl[...], lng[...], lnb[...])
    g_ref[0] = g


def _tc_transformer(h, bnn, tf, wl, bl, lng, lnb, pn_g, pn_b):
    xb = h.reshape(B, MAXLEN, D)
    r2 = lambda a: a.reshape(1, -1)
    full = lambda *s: pl.BlockSpec(s, lambda i: (0,) * len(s))
    args = [
        xb, r2(pn_g), r2(pn_b),
        tf["Wq"], r2(tf["bq"]), tf["Wk"], r2(tf["bk"]),
        tf["Wv"], r2(tf["bv"]), tf["Wo"], r2(tf["bo"]),
        tf["Wf1"], r2(tf["bf1"]), tf["Wf2"], r2(tf["bf2"]),
        r2(tf["ln1_g"]), r2(tf["ln1_b"]), r2(tf["ln2_g"]), r2(tf["ln2_b"]),
        wl, r2(bl), r2(lng), r2(lnb),
    ]
    in_specs = [pl.BlockSpec(memory_space=pltpu.SMEM),
                pl.BlockSpec((1, MAXLEN, D), lambda i: (i, 0, 0))]
    in_specs += [full(*a.shape) for a in args[1:]]
    lf, g = pl.pallas_call(
        _tf_body,
        grid=(B,),
        in_specs=in_specs,
        out_specs=[pl.BlockSpec((1, MAXLEN, D), lambda i: (i, 0, 0)),
                   pl.BlockSpec((1, 1, D), lambda i: (i, 0, 0))],
        out_shape=[
            jax.ShapeDtypeStruct((B, MAXLEN, D), jnp.float32),
            jax.ShapeDtypeStruct((B, 1, D), jnp.float32),
        ],
    )(bnn, *args)
    return lf.reshape(N, D), g.reshape(B, D)


_EA = 83200


def kernel(h, he, edge_index, batch_num_nodes, params):
    src = edge_index[0]
    dst = edge_index[1]
    halves = [
        (src[:_EA], dst[:_EA], he[:_EA]),
        (src[_EA:], dst[_EA:], he[_EA:]),
    ]
    zero_agg = jnp.zeros((2, N, 128), jnp.float32)

    pad_b = lambda w: jnp.pad(w, ((0, 0), (0, 128 - MR * DE)))
    wb_next = pad_b(params["edge1"]["We1"][D:2 * D])
    btab, h_bf = _tc_btab(h, pad_b(params["edge0"]["We1"][D:2 * D]))

    for l in range(NUM_LAYERS):
        pe = params["edge0" if l == 0 else "edge1"]
        pn = params["node0" if l == 0 else "node1"]
        wa = pe["We1"][:D]
        wc = pe["We1"][2 * D:]
        we2 = pe["We2"]
        wm1h = pn["Wm1"][:D]
        wm1e = pn["Wm1"][D:]
        wu1h = pn["Wu1"][:D]
        wu1a = pn["Wu1"][D:]

        msgs = []
        new_halves = []
        for (src_h, dst_h, he_h) in halves:
            hs = _sc_gather(h_bf, src_h, chunk=200)
            bd = _sc_gather(btab, dst_h, chunk=200)
            he2, msg = _tc_edge_msg(hs, bd, he_h, wa, wc, we2=we2,
                                    wm1h=wm1h, wm1e=wm1e, wm2=pn["Wm2"])
            new_halves.append((src_h, dst_h, he2))
            msgs.append(msg)
        halves = new_halves
        agg = _sc_scatter_add(msgs[0], halves[0][1], zero_agg)
        agg = _sc_scatter_add(msgs[1], halves[1][1], agg)
        h, h_bf, btab = _tc_node_update(h, agg, wu1h, wu1a, pn["Wu2"],
                                        wb_next)

    local_feat, global_feat = _tc_transformer(
        h, batch_num_nodes, params["tf"], params["Wl"], params["bl"],
        params["ln_g"], params["ln_b"], params["pn_g"], params["pn_b"])
    return local_feat, global_feat


# speedup vs baseline: 1.3778x; 1.0240x over previous
import functools

import jax
import jax.numpy as jnp
import numpy as np
from jax import lax
from jax.experimental import pallas as pl
from jax.experimental.pallas import tpu as pltpu
from jax.experimental.pallas import tpu_sc as plsc

N = 10000
E = 160000
B = 16
MAXLEN = 625
D = 256
DE = 16
MR = 4
H = 4
NUM_LAYERS = 4
DELTA = 0.1

_NC = 2
_NS = 16
_NW = _NC * _NS


def _bdot(a, b, out=jnp.float32):
    return jnp.dot(a.astype(jnp.bfloat16), b.astype(jnp.bfloat16),
                   preferred_element_type=out)


def _pack_bf16_pairs(x):
    bits = lax.bitcast_convert_type(x, jnp.int32)
    r = (bits + 0x7FFF + ((bits >> 16) & 1)) >> 16
    return (r[:, :128] & 0xFFFF) | (r[:, 128:] << 16)


def _unpack_bf16_pairs(p):
    lo = lax.bitcast_convert_type(p << 16, jnp.float32)
    hi = lax.bitcast_convert_type(p & jnp.int32(-65536), jnp.float32)
    return lo, hi


def _ln(x, g=None, b=None):
    m = jnp.mean(x, axis=-1, keepdims=True)
    v = jnp.mean((x - m) ** 2, axis=-1, keepdims=True)
    y = (x - m) * lax.rsqrt(v + 1e-5)
    if g is not None:
        y = y * g + b
    return y


def _gather_body(table_hbm, idx_hbm, out_hbm, idx_v, rows0, rows1,
                 gs0, gs1, ws0, ws1, *, per_w, chunk, n_ch):
    wid = lax.axis_index("s") * _NC + lax.axis_index("c")
    base = wid * per_w
    pltpu.sync_copy(idx_hbm.at[pl.ds(base, per_w)], idx_v)
    rows = (rows0, rows1)
    gsem = (gs0, gs1)
    wsem = (ws0, ws1)

    def start_gather(j):
        return pltpu.async_copy(
            table_hbm.at[idx_v.at[pl.ds(j * chunk, chunk)]],
            rows[j % 2], gsem[j % 2])

    gh = {0: start_gather(0)}
    wh = {}
    for j in range(n_ch):
        if j >= 1:
            wh[j - 1].wait()
        if j + 1 < n_ch:
            gh[j + 1] = start_gather(j + 1)
        gh[j].wait()
        off = pl.multiple_of(base + j * chunk, 8)
        wh[j] = pltpu.async_copy(rows[j % 2], out_hbm.at[pl.ds(off, chunk)],
                                 wsem[j % 2])
    wh[n_ch - 1].wait()


def _gather2_body(t1_hbm, i1_hbm, t2_hbm, i2_hbm, o1_hbm, o2_hbm,
                  idx1_v, idx2_v, r10, r11, r20, r21, gs0, gs1, ws0, ws1,
                  g2s0, g2s1, w2s0, w2s1, *, per_w, chunk, n_ch):
    wid = lax.axis_index("s") * _NC + lax.axis_index("c")
    base = wid * per_w
    pltpu.sync_copy(i1_hbm.at[pl.ds(base, per_w)], idx1_v)
    pltpu.sync_copy(i2_hbm.at[pl.ds(base, per_w)], idx2_v)
    tabs = ((t1_hbm, o1_hbm, (r10, r11), (gs0, gs1), (ws0, ws1), idx1_v),
            (t2_hbm, o2_hbm, (r20, r21), (g2s0, g2s1), (w2s0, w2s1), idx2_v))

    def start_gather(k, j):
        t, _, rows, gsem, _, iv = tabs[k]
        return pltpu.async_copy(
            t.at[iv.at[pl.ds(j * chunk, chunk)]],
            rows[j % 2], gsem[j % 2])

    work = [(k, j) for j in range(n_ch) for k in (0, 1)]
    gh = {}
    wh = {}
    for w in work[:2]:
        gh[w] = start_gather(*w)
    for i, (k, j) in enumerate(work):
        if i >= 2:
            wh[work[i - 2]].wait()
        if i + 2 < len(work):
            gh[work[i + 2]] = start_gather(*work[i + 2])
        gh[(k, j)].wait()
        _, o, rows, _, wsem, _ = tabs[k]
        off = pl.multiple_of(base + j * chunk, 8)
        wh[(k, j)] = pltpu.async_copy(rows[j % 2], o.at[pl.ds(off, chunk)],
                                      wsem[j % 2])
    wh[work[-2]].wait()
    wh[work[-1]].wait()


def _sc_gather2(t1, i1, t2, i2, chunk):
    e = i1.shape[0]
    per_w = e // _NW
    assert per_w % chunk == 0 and chunk % 8 == 0
    n_ch = per_w // chunk
    mesh = plsc.VectorSubcoreMesh(core_axis_name="c", subcore_axis_name="s")
    f = pl.kernel(
        functools.partial(_gather2_body, per_w=per_w, chunk=chunk, n_ch=n_ch),
        mesh=mesh,
        out_type=[jax.ShapeDtypeStruct((e, t1.shape[1]), t1.dtype),
                  jax.ShapeDtypeStruct((e, t2.shape[1]), t2.dtype)],
        scratch_types=[
            pltpu.VMEM((per_w,), jnp.int32),
            pltpu.VMEM((per_w,), jnp.int32),
            pltpu.VMEM((chunk, t1.shape[1]), t1.dtype),
            pltpu.VMEM((chunk, t1.shape[1]), t1.dtype),
            pltpu.VMEM((chunk, t2.shape[1]), t2.dtype),
            pltpu.VMEM((chunk, t2.shape[1]), t2.dtype),
        ] + [pltpu.SemaphoreType.DMA] * 8,
    )
    return f(t1, i1, t2, i2)


def _sc_gather(table, idx, chunk):
    rows, dt = table.shape
    e = idx.shape[0]
    per_w = e // _NW
    assert per_w * _NW == e and per_w % chunk == 0 and chunk % 8 == 0
    n_ch = per_w // chunk
    mesh = plsc.VectorSubcoreMesh(core_axis_name="c", subcore_axis_name="s")
    f = pl.kernel(
        functools.partial(_gather_body, per_w=per_w, chunk=chunk, n_ch=n_ch),
        mesh=mesh,
        out_type=jax.ShapeDtypeStruct((e, dt), table.dtype),
        scratch_types=[
            pltpu.VMEM((per_w,), jnp.int32),
            pltpu.VMEM((chunk, dt), table.dtype),
            pltpu.VMEM((chunk, dt), table.dtype),
            pltpu.SemaphoreType.DMA,
            pltpu.SemaphoreType.DMA,
            pltpu.SemaphoreType.DMA,
            pltpu.SemaphoreType.DMA,
        ],
    )
    return f(table, idx)


def _scatter_body(msg_hbm, dst_hbm, init_hbm, out_hbm, idx0, idx1, buf0,
                  buf1, acc, is0, is1, ms0, ms1, as0, as1,
                  *, ec, n_ch, per_tile, rows_per_tile):
    cid = lax.axis_index("c")
    sid = lax.axis_index("s")
    nbase = pl.multiple_of(sid * 624, 8)
    pltpu.sync_copy(init_hbm.at[cid, pl.ds(nbase, 640)],
                    acc.at[pl.ds(nbase, 640)])
    plsc.subcore_barrier()
    ebase = sid * per_tile
    idx = (idx0, idx1)
    buf = (buf0, buf1)
    isem = (is0, is1)
    msem = (ms0, ms1)
    asem = (as0, as1)

    def start_loads(j):
        off = pl.multiple_of(ebase + j * ec, 8)
        return (pltpu.async_copy(dst_hbm.at[pl.ds(off, ec)], idx[j % 2],
                                 isem[j % 2]),
                pltpu.async_copy(msg_hbm.at[cid, pl.ds(off, ec)], buf[j % 2],
                                 msem[j % 2]))

    hs = {0: start_loads(0)}
    ah = {}
    for j in range(n_ch):
        if j >= 2:
            ah[j - 2].wait()
        if j + 1 < n_ch:
            hs[j + 1] = start_loads(j + 1)
        hs[j][0].wait()
        hs[j][1].wait()
        ah[j] = pltpu.async_copy(buf[j % 2], acc.at[idx[j % 2]], asem[j % 2],
                                 add=True)
    ah[n_ch - 2].wait()
    ah[n_ch - 1].wait()
    plsc.subcore_barrier()
    pltpu.sync_copy(acc.at[pl.ds(nbase, 640)],
                    out_hbm.at[cid, pl.ds(nbase, 640)])


def _sc_scatter_add(msg3, dst, init, ec=80):
    e = msg3.shape[1]
    per_tile = e // _NS
    rows_per_tile = N // _NS
    assert per_tile % ec == 0 and ec % 8 == 0
    assert 624 * (_NS - 1) + 640 == N
    n_ch = per_tile // ec
    mesh = plsc.VectorSubcoreMesh(core_axis_name="c", subcore_axis_name="s")
    f = pl.kernel(
        functools.partial(_scatter_body, ec=ec, n_ch=n_ch, per_tile=per_tile,
                          rows_per_tile=rows_per_tile),
        mesh=mesh,
        out_type=jax.ShapeDtypeStruct((2, N, 128), jnp.float32),
        scratch_types=[
            pltpu.VMEM((ec,), jnp.int32),
            pltpu.VMEM((ec,), jnp.int32),
            pltpu.VMEM((ec, 128), jnp.float32),
            pltpu.VMEM((ec, 128), jnp.float32),
            pltpu.VMEM_SHARED((N, 128), jnp.float32),
            pltpu.SemaphoreType.DMA,
            pltpu.SemaphoreType.DMA,
            pltpu.SemaphoreType.DMA,
            pltpu.SemaphoreType.DMA,
            pltpu.SemaphoreType.DMA,
            pltpu.SemaphoreType.DMA,
        ],
    )
    return f(msg3, dst, init)


_EB = 1600


def _edge_msg_body(hs_ref, bd_ref, he_ref, wa, wc, we2, wm1h, wm1e, wm2,
                   he_out, msg_out):
    hs_lo, hs_hi = _unpack_bf16_pairs(hs_ref[...])
    hs = jnp.concatenate([hs_lo, hs_hi], axis=-1)
    he = he_ref[...]
    f32 = jnp.float32
    z = (jnp.dot(hs, wa[...], preferred_element_type=f32)
         + bd_ref[...][:, :MR * DE]
         + jnp.dot(he, wc[...], preferred_element_type=f32))
    m = jnp.dot(jax.nn.relu(z), we2[...], preferred_element_type=f32)
    he2 = _ln(he + m)
    he_out[...] = he2
    p = jax.nn.relu(
        jnp.dot(hs, wm1h[...], preferred_element_type=f32)
        + jnp.dot(he2, wm1e[...], preferred_element_type=f32))
    msgv = jnp.dot(p, wm2[...], preferred_element_type=f32)
    msg_out[0] = msgv[:, :128]
    msg_out[1] = msgv[:, 128:]


def _tc_edge_msg(hs, bd, he, wa, wc, we2, wm1h, wm1e, wm2):
    e = hs.shape[0]
    grid = (e // _EB,)
    full = lambda *s: pl.BlockSpec(s, lambda i: (0,) * len(s))
    row = lambda *s: pl.BlockSpec(s, lambda i: (i,) + (0,) * (len(s) - 1))
    return pl.pallas_call(
        _edge_msg_body,
        grid=grid,
        in_specs=[
            row(_EB, 128), row(_EB, 128), row(_EB, DE),
            full(D, MR * DE), full(DE, MR * DE), full(MR * DE, DE),
            full(D, MR * D), full(DE, MR * D), full(MR * D, D),
        ],
        out_specs=[row(_EB, DE),
                   pl.BlockSpec((2, _EB, 128), lambda i: (0, i, 0))],
        out_shape=[
            jax.ShapeDtypeStruct((e, DE), jnp.float32),
            jax.ShapeDtypeStruct((2, e, 128), jnp.float32),
        ],
    )(hs, bd, he, wa, wc, we2, wm1h, wm1e, wm2)


_NB = 2000


def _node_body(h_ref, agg_ref, wu1h, wu1a, wu2, wbn, h_out, hb_out, btab_out):
    h = h_ref[...]
    f32 = jnp.float32
    u0 = jax.nn.relu(jnp.dot(h, wu1h[...], preferred_element_type=f32)
                     + jnp.dot(agg_ref[0], wu1a[...][:128],
                               preferred_element_type=f32)
                     + jnp.dot(agg_ref[1], wu1a[...][128:],
                               preferred_element_type=f32))
    u = jnp.dot(u0, wu2[...], preferred_element_type=f32)
    h2 = _ln(h + DELTA * u)
    h_out[...] = h2
    hb_out[...] = _pack_bf16_pairs(h2)
    btab_out[...] = jnp.dot(h2, wbn[...], preferred_element_type=jnp.float32)


def _tc_node_update(h, agg, wu1h, wu1a, wu2, wbn):
    grid = (N // _NB,)
    full = lambda *s: pl.BlockSpec(s, lambda i: (0,) * len(s))
    row = lambda *s: pl.BlockSpec(s, lambda i: (i,) + (0,) * (len(s) - 1))
    return pl.pallas_call(
        _node_body,
        grid=grid,
        in_specs=[
            row(_NB, D),
            pl.BlockSpec((2, _NB, 128), lambda i: (0, i, 0)),
            full(D, MR * D), full(D, MR * D), full(MR * D, D),
            full(D, 128),
        ],
        out_specs=[row(_NB, D), row(_NB, 128), row(_NB, 128)],
        out_shape=[
            jax.ShapeDtypeStruct((N, D), jnp.float32),
            jax.ShapeDtypeStruct((N, 128), jnp.int32),
            jax.ShapeDtypeStruct((N, 128), jnp.float32),
        ],
    )(h, agg, wu1h, wu1a, wu2, wbn)


def _btab_body(h_ref, wb, out_ref, hb_ref):
    out_ref[...] = jnp.dot(h_ref[...], wb[...],
                           preferred_element_type=jnp.float32)
    hb_ref[...] = _pack_bf16_pairs(h_ref[...])


def _tc_btab(h, wb):
    return pl.pallas_call(
        _btab_body,
        grid=(N // _NB,),
        in_specs=[pl.BlockSpec((_NB, D), lambda i: (i, 0)),
                  pl.BlockSpec((D, 128), lambda i: (0, 0))],
        out_specs=[pl.BlockSpec((_NB, 128), lambda i: (i, 0)),
                   pl.BlockSpec((_NB, 128), lambda i: (i, 0))],
        out_shape=[jax.ShapeDtypeStruct((N, 128), jnp.float32),
                   jax.ShapeDtypeStruct((N, 128), jnp.int32)],
    )(h, wb)


def _tf_body(bnn_ref, x_ref, pn_g, pn_b, wq, bq, wk, bk, wv, bv, wo, bo,
             wf1, bf1, wf2, bf2, l1g, l1b, l2g, l2b, wl, bl, lng, lnb,
             lf_ref, g_ref):
    bidx = pl.program_id(0)
    nb = bnn_ref[bidx]
    x0 = _ln(x_ref[0], pn_g[...], pn_b[...])
    q = _bdot(x0, wq[...]) + bq[...]
    k = _bdot(x0, wk[...]) + bk[...]
    v = _bdot(x0, wv[...]) + bv[...]
    colmask = lax.broadcasted_iota(jnp.int32, (MAXLEN, MAXLEN), 1) >= nb
    dh = D // H
    outs = []
    for hh in range(H):
        sl = slice(hh * dh, (hh + 1) * dh)
        s = lax.dot_general(q[:, sl].astype(jnp.bfloat16),
                            k[:, sl].astype(jnp.bfloat16),
                            (((1,), (1,)), ((), ())),
                            preferred_element_type=jnp.float32)
        s = s * np.float32(1.0 / np.sqrt(dh))
        s = jnp.where(colmask, np.float32(-1e9), s)
        s = s - jnp.max(s, axis=-1, keepdims=True)
        es = jnp.exp(s)
        a = es / jnp.sum(es, axis=-1, keepdims=True)
        outs.append(_bdot(a, v[:, sl]))
    o = jnp.concatenate(outs, axis=1)
    o = _bdot(o, wo[...]) + bo[...]
    x1 = _ln(x0 + o, l1g[...], l1b[...])
    f0 = jax.nn.relu(_bdot(x1, wf1[...]) + bf1[...])
    f = _bdot(f0, wf2[...]) + bf2[...]
    lf = _ln(x1 + f, l2g[...], l2b[...])
    lf_ref[0] = lf
    rowmask = lax.broadcasted_iota(jnp.int32, (MAXLEN, 1), 0) < nb
    pooled = (jnp.sum(jnp.where(rowmask, lf, 0.0), axis=0, keepdims=True)
              / nb.astype(jnp.float32))
    g = _ln(_bdot(pooled, wl[...]) + bl[...], lng[...], lnb[...])
    g_ref[0] = g


def _tc_transformer(h, bnn, tf, wl, bl, lng, lnb, pn_g, pn_b):
    xb = h.reshape(B, MAXLEN, D)
    r2 = lambda a: a.reshape(1, -1)
    full = lambda *s: pl.BlockSpec(s, lambda i: (0,) * len(s))
    args = [
        xb, r2(pn_g), r2(pn_b),
        tf["Wq"], r2(tf["bq"]), tf["Wk"], r2(tf["bk"]),
        tf["Wv"], r2(tf["bv"]), tf["Wo"], r2(tf["bo"]),
        tf["Wf1"], r2(tf["bf1"]), tf["Wf2"], r2(tf["bf2"]),
        r2(tf["ln1_g"]), r2(tf["ln1_b"]), r2(tf["ln2_g"]), r2(tf["ln2_b"]),
        wl, r2(bl), r2(lng), r2(lnb),
    ]
    in_specs = [pl.BlockSpec(memory_space=pltpu.SMEM),
                pl.BlockSpec((1, MAXLEN, D), lambda i: (i, 0, 0))]
    in_specs += [full(*a.shape) for a in args[1:]]
    lf, g = pl.pallas_call(
        _tf_body,
        grid=(B,),
        in_specs=in_specs,
        out_specs=[pl.BlockSpec((1, MAXLEN, D), lambda i: (i, 0, 0)),
                   pl.BlockSpec((1, 1, D), lambda i: (i, 0, 0))],
        out_shape=[
            jax.ShapeDtypeStruct((B, MAXLEN, D), jnp.float32),
            jax.ShapeDtypeStruct((B, 1, D), jnp.float32),
        ],
    )(bnn, *args)
    return lf.reshape(N, D), g.reshape(B, D)


_EA = 83200


def kernel(h, he, edge_index, batch_num_nodes, params):
    src = edge_index[0]
    dst = edge_index[1]
    halves = [
        (src[:_EA], dst[:_EA], he[:_EA]),
        (src[_EA:], dst[_EA:], he[_EA:]),
    ]
    zero_agg = jnp.zeros((2, N, 128), jnp.float32)

    pad_b = lambda w: jnp.pad(w, ((0, 0), (0, 128 - MR * DE)))
    wb_next = pad_b(params["edge1"]["We1"][D:2 * D])
    btab, h_bf = _tc_btab(h, pad_b(params["edge0"]["We1"][D:2 * D]))

    for l in range(NUM_LAYERS):
        pe = params["edge0" if l == 0 else "edge1"]
        pn = params["node0" if l == 0 else "node1"]
        wa = pe["We1"][:D]
        wc = pe["We1"][2 * D:]
        we2 = pe["We2"]
        wm1h = pn["Wm1"][:D]
        wm1e = pn["Wm1"][D:]
        wu1h = pn["Wu1"][:D]
        wu1a = pn["Wu1"][D:]

        msgs = []
        new_halves = []
        for (src_h, dst_h, he_h) in halves:
            hs, bd = _sc_gather2(h_bf, src_h, btab, dst_h, chunk=200)
            he2, msg = _tc_edge_msg(hs, bd, he_h, wa, wc, we2=we2,
                                    wm1h=wm1h, wm1e=wm1e, wm2=pn["Wm2"])
            new_halves.append((src_h, dst_h, he2))
            msgs.append(msg)
        halves = new_halves
        agg = _sc_scatter_add(msgs[0], halves[0][1], zero_agg)
        agg = _sc_scatter_add(msgs[1], halves[1][1], agg)
        h, h_bf, btab = _tc_node_update(h, agg, wu1h, wu1a, pn["Wu2"],
                                        wb_next)

    local_feat, global_feat = _tc_transformer(
        h, batch_num_nodes, params["tf"], params["Wl"], params["bl"],
        params["ln_g"], params["ln_b"], params["pn_g"], params["pn_b"])
    return local_feat, global_feat


# speedup vs baseline: 1.3840x; 1.0045x over previous
import functools

import jax
import jax.numpy as jnp
import numpy as np
from jax import lax
from jax.experimental import pallas as pl
from jax.experimental.pallas import tpu as pltpu
from jax.experimental.pallas import tpu_sc as plsc

N = 10000
E = 160000
B = 16
MAXLEN = 625
D = 256
DE = 16
MR = 4
H = 4
NUM_LAYERS = 4
DELTA = 0.1

_NC = 2
_NS = 16
_NW = _NC * _NS


def _bdot(a, b, out=jnp.float32):
    return jnp.dot(a.astype(jnp.bfloat16), b.astype(jnp.bfloat16),
                   preferred_element_type=out)


def _pack_bf16_pairs(x):
    bits = lax.bitcast_convert_type(x, jnp.int32)
    r = (bits + 0x7FFF + ((bits >> 16) & 1)) >> 16
    return (r[:, :128] & 0xFFFF) | (r[:, 128:] << 16)


def _unpack_bf16_pairs(p):
    lo = lax.bitcast_convert_type(p << 16, jnp.float32)
    hi = lax.bitcast_convert_type(p & jnp.int32(-65536), jnp.float32)
    return lo, hi


def _ln(x, g=None, b=None):
    m = jnp.mean(x, axis=-1, keepdims=True)
    v = jnp.mean((x - m) ** 2, axis=-1, keepdims=True)
    y = (x - m) * lax.rsqrt(v + 1e-5)
    if g is not None:
        y = y * g + b
    return y


def _gather_body(table_hbm, idx_hbm, out_hbm, idx_v, rows0, rows1,
                 gs0, gs1, ws0, ws1, *, per_w, chunk, n_ch):
    wid = lax.axis_index("s") * _NC + lax.axis_index("c")
    base = wid * per_w
    pltpu.sync_copy(idx_hbm.at[pl.ds(base, per_w)], idx_v)
    rows = (rows0, rows1)
    gsem = (gs0, gs1)
    wsem = (ws0, ws1)

    def start_gather(j):
        return pltpu.async_copy(
            table_hbm.at[idx_v.at[pl.ds(j * chunk, chunk)]],
            rows[j % 2], gsem[j % 2])

    gh = {0: start_gather(0)}
    wh = {}
    for j in range(n_ch):
        if j >= 1:
            wh[j - 1].wait()
        if j + 1 < n_ch:
            gh[j + 1] = start_gather(j + 1)
        gh[j].wait()
        off = pl.multiple_of(base + j * chunk, 8)
        wh[j] = pltpu.async_copy(rows[j % 2], out_hbm.at[pl.ds(off, chunk)],
                                 wsem[j % 2])
    wh[n_ch - 1].wait()


def _gather2_body(t1_hbm, i1_hbm, t2_hbm, i2_hbm, o1_hbm, o2_hbm,
                  idx1_v, idx2_v, r10, r11, r20, r21, gs0, gs1, ws0, ws1,
                  g2s0, g2s1, w2s0, w2s1, *, per_w, chunk, n_ch):
    wid = lax.axis_index("s") * _NC + lax.axis_index("c")
    base = wid * per_w
    pltpu.sync_copy(i1_hbm.at[pl.ds(base, per_w)], idx1_v)
    pltpu.sync_copy(i2_hbm.at[pl.ds(base, per_w)], idx2_v)
    tabs = ((t1_hbm, o1_hbm, (r10, r11), (gs0, gs1), (ws0, ws1), idx1_v),
            (t2_hbm, o2_hbm, (r20, r21), (g2s0, g2s1), (w2s0, w2s1), idx2_v))

    def start_gather(k, j):
        t, _, rows, gsem, _, iv = tabs[k]
        return pltpu.async_copy(
            t.at[iv.at[pl.ds(j * chunk, chunk)]],
            rows[j % 2], gsem[j % 2])

    work = [(k, j) for j in range(n_ch) for k in (0, 1)]
    gh = {}
    wh = {}
    for w in work[:2]:
        gh[w] = start_gather(*w)
    for i, (k, j) in enumerate(work):
        if i >= 2:
            wh[work[i - 2]].wait()
        if i + 2 < len(work):
            gh[work[i + 2]] = start_gather(*work[i + 2])
        gh[(k, j)].wait()
        _, o, rows, _, wsem, _ = tabs[k]
        off = pl.multiple_of(base + j * chunk, 8)
        wh[(k, j)] = pltpu.async_copy(rows[j % 2], o.at[pl.ds(off, chunk)],
                                      wsem[j % 2])
    wh[work[-2]].wait()
    wh[work[-1]].wait()


def _sc_gather2(t1, i1, t2, i2, chunk):
    e = i1.shape[0]
    per_w = e // _NW
    assert per_w % chunk == 0 and chunk % 8 == 0
    n_ch = per_w // chunk
    mesh = plsc.VectorSubcoreMesh(core_axis_name="c", subcore_axis_name="s")
    f = pl.kernel(
        functools.partial(_gather2_body, per_w=per_w, chunk=chunk, n_ch=n_ch),
        mesh=mesh,
        out_type=[jax.ShapeDtypeStruct((e, t1.shape[1]), t1.dtype),
                  jax.ShapeDtypeStruct((e, t2.shape[1]), t2.dtype)],
        scratch_types=[
            pltpu.VMEM((per_w,), jnp.int32),
            pltpu.VMEM((per_w,), jnp.int32),
            pltpu.VMEM((chunk, t1.shape[1]), t1.dtype),
            pltpu.VMEM((chunk, t1.shape[1]), t1.dtype),
            pltpu.VMEM((chunk, t2.shape[1]), t2.dtype),
            pltpu.VMEM((chunk, t2.shape[1]), t2.dtype),
        ] + [pltpu.SemaphoreType.DMA] * 8,
    )
    return f(t1, i1, t2, i2)


def _sc_gather(table, idx, chunk):
    rows, dt = table.shape
    e = idx.shape[0]
    per_w = e // _NW
    assert per_w * _NW == e and per_w % chunk == 0 and chunk % 8 == 0
    n_ch = per_w // chunk
    mesh = plsc.VectorSubcoreMesh(core_axis_name="c", subcore_axis_name="s")
    f = pl.kernel(
        functools.partial(_gather_body, per_w=per_w, chunk=chunk, n_ch=n_ch),
        mesh=mesh,
        out_type=jax.ShapeDtypeStruct((e, dt), table.dtype),
        scratch_types=[
            pltpu.VMEM((per_w,), jnp.int32),
            pltpu.VMEM((chunk, dt), table.dtype),
            pltpu.VMEM((chunk, dt), table.dtype),
            pltpu.SemaphoreType.DMA,
            pltpu.SemaphoreType.DMA,
            pltpu.SemaphoreType.DMA,
            pltpu.SemaphoreType.DMA,
        ],
    )
    return f(table, idx)


def _scatter_body(msg_hbm, dst_hbm, init_hbm, out_hbm, idx0, idx1, buf0,
                  buf1, acc, is0, is1, ms0, ms1, as0, as1,
                  *, ec, n_ch, per_tile, rows_per_tile):
    cid = lax.axis_index("c")
    sid = lax.axis_index("s")
    nbase = pl.multiple_of(sid * 624, 8)
    pltpu.sync_copy(init_hbm.at[cid, pl.ds(nbase, 640)],
                    acc.at[pl.ds(nbase, 640)])
    plsc.subcore_barrier()
    ebase = sid * per_tile
    idx = (idx0, idx1)
    buf = (buf0, buf1)
    isem = (is0, is1)
    msem = (ms0, ms1)
    asem = (as0, as1)

    def start_loads(j):
        off = pl.multiple_of(ebase + j * ec, 8)
        return (pltpu.async_copy(dst_hbm.at[pl.ds(off, ec)], idx[j % 2],
                                 isem[j % 2]),
                pltpu.async_copy(msg_hbm.at[cid, pl.ds(off, ec)], buf[j % 2],
                                 msem[j % 2]))

    hs = {0: start_loads(0)}
    ah = {}
    for j in range(n_ch):
        if j >= 2:
            ah[j - 2].wait()
        if j + 1 < n_ch:
            hs[j + 1] = start_loads(j + 1)
        hs[j][0].wait()
        hs[j][1].wait()
        ah[j] = pltpu.async_copy(buf[j % 2], acc.at[idx[j % 2]], asem[j % 2],
                                 add=True)
    ah[n_ch - 2].wait()
    ah[n_ch - 1].wait()
    plsc.subcore_barrier()
    pltpu.sync_copy(acc.at[pl.ds(nbase, 640)],
                    out_hbm.at[cid, pl.ds(nbase, 640)])


def _sc_scatter_add(msg3, dst, init, ec=80):
    e = msg3.shape[1]
    per_tile = e // _NS
    rows_per_tile = N // _NS
    assert per_tile % ec == 0 and ec % 8 == 0
    assert 624 * (_NS - 1) + 640 == N
    n_ch = per_tile // ec
    mesh = plsc.VectorSubcoreMesh(core_axis_name="c", subcore_axis_name="s")
    f = pl.kernel(
        functools.partial(_scatter_body, ec=ec, n_ch=n_ch, per_tile=per_tile,
                          rows_per_tile=rows_per_tile),
        mesh=mesh,
        out_type=jax.ShapeDtypeStruct((2, N, 128), jnp.float32),
        scratch_types=[
            pltpu.VMEM((ec,), jnp.int32),
            pltpu.VMEM((ec,), jnp.int32),
            pltpu.VMEM((ec, 128), jnp.float32),
            pltpu.VMEM((ec, 128), jnp.float32),
            pltpu.VMEM_SHARED((N, 128), jnp.float32),
            pltpu.SemaphoreType.DMA,
            pltpu.SemaphoreType.DMA,
            pltpu.SemaphoreType.DMA,
            pltpu.SemaphoreType.DMA,
            pltpu.SemaphoreType.DMA,
            pltpu.SemaphoreType.DMA,
        ],
    )
    return f(msg3, dst, init)


_EB = 1600


def _edge_msg_body(hs_ref, bd_ref, he_ref, wa, wc, we2, wm1h, wm1e, wm2,
                   he_out, msg_out):
    hs_lo, hs_hi = _unpack_bf16_pairs(hs_ref[...])
    hs = jnp.concatenate([hs_lo, hs_hi], axis=-1)
    he = he_ref[...]
    f32 = jnp.float32
    z = (jnp.dot(hs, wa[...], preferred_element_type=f32)
         + bd_ref[...][:, :MR * DE]
         + jnp.dot(he, wc[...], preferred_element_type=f32))
    m = jnp.dot(jax.nn.relu(z), we2[...], preferred_element_type=f32)
    he2 = _ln(he + m)
    he_out[...] = he2
    p = jax.nn.relu(
        jnp.dot(hs, wm1h[...], preferred_element_type=f32)
        + jnp.dot(he2, wm1e[...], preferred_element_type=f32))
    msgv = jnp.dot(p, wm2[...], preferred_element_type=f32)
    msg_out[0] = msgv[:, :128]
    msg_out[1] = msgv[:, 128:]


def _tc_edge_msg(hs, bd, he, wa, wc, we2, wm1h, wm1e, wm2, he_off=0):
    e = hs.shape[0]
    grid = (e // _EB,)
    full = lambda *s: pl.BlockSpec(s, lambda i: (0,) * len(s))
    row = lambda *s: pl.BlockSpec(s, lambda i: (i,) + (0,) * (len(s) - 1))
    return pl.pallas_call(
        _edge_msg_body,
        grid=grid,
        in_specs=[
            row(_EB, 128), row(_EB, 128),
            pl.BlockSpec((_EB, DE), lambda i: (i + he_off, 0)),
            full(D, MR * DE), full(DE, MR * DE), full(MR * DE, DE),
            full(D, MR * D), full(DE, MR * D), full(MR * D, D),
        ],
        out_specs=[row(_EB, DE),
                   pl.BlockSpec((2, _EB, 128), lambda i: (0, i, 0))],
        out_shape=[
            jax.ShapeDtypeStruct((e, DE), jnp.float32),
            jax.ShapeDtypeStruct((2, e, 128), jnp.float32),
        ],
    )(hs, bd, he, wa, wc, we2, wm1h, wm1e, wm2)


_NB = 2000


def _node_body(h_ref, agg_ref, wu1h, wu1a, wu2, wbn, h_out, hb_out, btab_out):
    h = h_ref[...]
    f32 = jnp.float32
    u0 = jax.nn.relu(jnp.dot(h, wu1h[...], preferred_element_type=f32)
                     + jnp.dot(agg_ref[0], wu1a[...][:128],
                               preferred_element_type=f32)
                     + jnp.dot(agg_ref[1], wu1a[...][128:],
                               preferred_element_type=f32))
    u = jnp.dot(u0, wu2[...], preferred_element_type=f32)
    h2 = _ln(h + DELTA * u)
    h_out[...] = h2
    hb_out[...] = _pack_bf16_pairs(h2)
    btab_out[...] = jnp.dot(h2, wbn[...], preferred_element_type=jnp.float32)


def _tc_node_update(h, agg, wu1h, wu1a, wu2, wbn):
    grid = (N // _NB,)
    full = lambda *s: pl.BlockSpec(s, lambda i: (0,) * len(s))
    row = lambda *s: pl.BlockSpec(s, lambda i: (i,) + (0,) * (len(s) - 1))
    return pl.pallas_call(
        _node_body,
        grid=grid,
        in_specs=[
            row(_NB, D),
            pl.BlockSpec((2, _NB, 128), lambda i: (0, i, 0)),
            full(D, MR * D), full(D, MR * D), full(MR * D, D),
            full(D, 128),
        ],
        out_specs=[row(_NB, D), row(_NB, 128), row(_NB, 128)],
        out_shape=[
            jax.ShapeDtypeStruct((N, D), jnp.float32),
            jax.ShapeDtypeStruct((N, 128), jnp.int32),
            jax.ShapeDtypeStruct((N, 128), jnp.float32),
        ],
    )(h, agg, wu1h, wu1a, wu2, wbn)


def _btab_body(h_ref, wb, out_ref, hb_ref):
    out_ref[...] = jnp.dot(h_ref[...], wb[...],
                           preferred_element_type=jnp.float32)
    hb_ref[...] = _pack_bf16_pairs(h_ref[...])


def _tc_btab(h, wb):
    return pl.pallas_call(
        _btab_body,
        grid=(N // _NB,),
        in_specs=[pl.BlockSpec((_NB, D), lambda i: (i, 0)),
                  pl.BlockSpec((D, 128), lambda i: (0, 0))],
        out_specs=[pl.BlockSpec((_NB, 128), lambda i: (i, 0)),
                   pl.BlockSpec((_NB, 128), lambda i: (i, 0))],
        out_shape=[jax.ShapeDtypeStruct((N, 128), jnp.float32),
                   jax.ShapeDtypeStruct((N, 128), jnp.int32)],
    )(h, wb)


def _tf_body(bnn_ref, x_ref, pn_g, pn_b, wq, bq, wk, bk, wv, bv, wo, bo,
             wf1, bf1, wf2, bf2, l1g, l1b, l2g, l2b, wl, bl, lng, lnb,
             lf_ref, g_ref):
    bidx = pl.program_id(0)
    nb = bnn_ref[bidx]
    x0 = _ln(x_ref[0], pn_g[...], pn_b[...])
    q = _bdot(x0, wq[...]) + bq[...]
    k = _bdot(x0, wk[...]) + bk[...]
    v = _bdot(x0, wv[...]) + bv[...]
    colmask = lax.broadcasted_iota(jnp.int32, (MAXLEN, MAXLEN), 1) >= nb
    dh = D // H
    outs = []
    for hh in range(H):
        sl = slice(hh * dh, (hh + 1) * dh)
        s = lax.dot_general(q[:, sl].astype(jnp.bfloat16),
                            k[:, sl].astype(jnp.bfloat16),
                            (((1,), (1,)), ((), ())),
                            preferred_element_type=jnp.float32)
        s = s * np.float32(1.0 / np.sqrt(dh))
        s = jnp.where(colmask, np.float32(-1e9), s)
        s = s - jnp.max(s, axis=-1, keepdims=True)
        es = jnp.exp(s)
        a = es / jnp.sum(es, axis=-1, keepdims=True)
        outs.append(_bdot(a, v[:, sl]))
    o = jnp.concatenate(outs, axis=1)
    o = _bdot(o, wo[...]) + bo[...]
    x1 = _ln(x0 + o, l1g[...], l1b[...])
    f0 = jax.nn.relu(_bdot(x1, wf1[...]) + bf1[...])
    f = _bdot(f0, wf2[...]) + bf2[...]
    lf = _ln(x1 + f, l2g[...], l2b[...])
    lf_ref[0] = lf
    rowmask = lax.broadcasted_iota(jnp.int32, (MAXLEN, 1), 0) < nb
    pooled = (jnp.sum(jnp.where(rowmask, lf, 0.0), axis=0, keepdims=True)
              / nb.astype(jnp.float32))
    g = _ln(_bdot(pooled, wl[...]) + bl[...], lng[...], lnb[...])
    g_ref[0] = g


def _tc_transformer(h, bnn, tf, wl, bl, lng, lnb, pn_g, pn_b):
    xb = h.reshape(B, MAXLEN, D)
    r2 = lambda a: a.reshape(1, -1)
    full = lambda *s: pl.BlockSpec(s, lambda i: (0,) * len(s))
    args = [
        xb, r2(pn_g), r2(pn_b),
        tf["Wq"], r2(tf["bq"]), tf["Wk"], r2(tf["bk"]),
        tf["Wv"], r2(tf["bv"]), tf["Wo"], r2(tf["bo"]),
        tf["Wf1"], r2(tf["bf1"]), tf["Wf2"], r2(tf["bf2"]),
        r2(tf["ln1_g"]), r2(tf["ln1_b"]), r2(tf["ln2_g"]), r2(tf["ln2_b"]),
        wl, r2(bl), r2(lng), r2(lnb),
    ]
    in_specs = [pl.BlockSpec(memory_space=pltpu.SMEM),
                pl.BlockSpec((1, MAXLEN, D), lambda i: (i, 0, 0))]
    in_specs += [full(*a.shape) for a in args[1:]]
    lf, g = pl.pallas_call(
        _tf_body,
        grid=(B,),
        in_specs=in_specs,
        out_specs=[pl.BlockSpec((1, MAXLEN, D), lambda i: (i, 0, 0)),
                   pl.BlockSpec((1, 1, D), lambda i: (i, 0, 0))],
        out_shape=[
            jax.ShapeDtypeStruct((B, MAXLEN, D), jnp.float32),
            jax.ShapeDtypeStruct((B, 1, D), jnp.float32),
        ],
    )(bnn, *args)
    return lf.reshape(N, D), g.reshape(B, D)


_EA = 83200


def kernel(h, he, edge_index, batch_num_nodes, params):
    src = edge_index[0]
    dst = edge_index[1]
    halves = [
        (src[:_EA], dst[:_EA], he, 0),
        (src[_EA:], dst[_EA:], he, _EA // _EB),
    ]
    zero_agg = jnp.zeros((2, N, 128), jnp.float32)

    pad_b = lambda w: jnp.pad(w, ((0, 0), (0, 128 - MR * DE)))
    wb_next = pad_b(params["edge1"]["We1"][D:2 * D])
    btab, h_bf = _tc_btab(h, pad_b(params["edge0"]["We1"][D:2 * D]))

    for l in range(NUM_LAYERS):
        pe = params["edge0" if l == 0 else "edge1"]
        pn = params["node0" if l == 0 else "node1"]
        wa = pe["We1"][:D]
        wc = pe["We1"][2 * D:]
        we2 = pe["We2"]
        wm1h = pn["Wm1"][:D]
        wm1e = pn["Wm1"][D:]
        wu1h = pn["Wu1"][:D]
        wu1a = pn["Wu1"][D:]

        msgs = []
        new_halves = []
        for (src_h, dst_h, he_h, he_off) in halves:
            hs, bd = _sc_gather2(h_bf, src_h, btab, dst_h, chunk=200)
            he2, msg = _tc_edge_msg(hs, bd, he_h, wa, wc, we2=we2,
                                    wm1h=wm1h, wm1e=wm1e, wm2=pn["Wm2"],
                                    he_off=he_off)
            new_halves.append((src_h, dst_h, he2, 0))
            msgs.append(msg)
        halves = new_halves
        agg = _sc_scatter_add(msgs[0], halves[0][1], zero_agg)
        agg = _sc_scatter_add(msgs[1], halves[1][1], agg)
        h, h_bf, btab = _tc_node_update(h, agg, wu1h, wu1a, pn["Wu2"],
                                        wb_next)

    local_feat, global_feat = _tc_transformer(
        h, batch_num_nodes, params["tf"], params["Wl"], params["bl"],
        params["ln_g"], params["ln_b"], params["pn_g"], params["pn_b"])
    return local_feat, global_feat
